# Initial kernel scaffold; baseline (speedup 1.0000x reference)
import jax
import jax.numpy as jnp
from jax.experimental import pallas as pl


def kernel(train_x, train_y, test_x):
    raise NotImplementedError("write your pallas kernel here")


# speedup vs baseline: 3.1715x; 3.1715x over previous
import functools

import jax
import jax.numpy as jnp
from jax import lax
from jax.experimental import pallas as pl
from jax.experimental.pallas import tpu as pltpu
from jax.experimental.pallas import tpu_sc as plsc

K = 16
B = 1024
D = 16
N = 100000
C = 2048
NCH = (N + C - 1) // C
NP = NCH * C

NW = 32
QW = B // NW
IW = QW * K


def _topk_body(trainT_ref, test_ref, idx_ref, tv_ref, ti_ref):
    c = pl.program_id(0)

    @pl.when(c == 0)
    def _init():
        tv_ref[:] = jnp.full((B, K), jnp.inf, jnp.float32)
        ti_ref[:] = jnp.zeros((B, K), jnp.int32)

    xc = trainT_ref[:]
    t = test_ref[:]
    tr = jnp.sum(xc * xc, axis=0, keepdims=True)
    te = jnp.sum(t * t, axis=1, keepdims=True)
    mm = jnp.dot(t, xc, preferred_element_type=jnp.float32)
    d = jnp.sqrt(jnp.maximum(te + tr - 2.0 * mm, 0.0))
    lane = lax.broadcasted_iota(jnp.int32, (B, C), 1)
    d = jnp.where(lane + c * C >= N, jnp.inf, d)

    m0 = jnp.min(d, axis=1, keepdims=True)

    def cond(state):
        tv, ti, m, p, it = state
        return jnp.logical_and(it < K, jnp.any(m < tv[:, K - 1:K]))

    def body(state):
        tv, ti, m, p, it = state
        pn = jnp.min(jnp.where(d == m, lane, NP), axis=1, keepdims=True)
        g = pn + c * C
        pos = jnp.sum((tv <= m).astype(jnp.int32), axis=1, keepdims=True)
        j16 = lax.broadcasted_iota(jnp.int32, (B, K), 1)
        tv_sh = jnp.concatenate([m, tv[:, :K - 1]], axis=1)
        ti_sh = jnp.concatenate([g, ti[:, :K - 1]], axis=1)
        keep = j16 < pos
        ins = j16 == pos
        tv2 = jnp.where(keep, tv, jnp.where(ins, jnp.broadcast_to(m, (B, K)), tv_sh))
        ti2 = jnp.where(keep, ti, jnp.where(ins, jnp.broadcast_to(g, (B, K)), ti_sh))
        nxt = (d > m) | ((d == m) & (lane > pn))
        m2 = jnp.min(jnp.where(nxt, d, jnp.inf), axis=1, keepdims=True)
        return tv2, ti2, m2, pn, it + 1

    tv, ti, _, _, _ = lax.while_loop(
        cond, body,
        (tv_ref[:], ti_ref[:], m0, jnp.full((B, 1), -1, jnp.int32), jnp.int32(0)))
    tv_ref[:] = tv
    ti_ref[:] = ti

    @pl.when(c == NCH - 1)
    def _out():
        idx_ref[:] = ti


def _topk_call(trainT_p, test_x):
    return pl.pallas_call(
        _topk_body,
        grid=(NCH,),
        in_specs=[
            pl.BlockSpec((D, C), lambda c: (0, c)),
            pl.BlockSpec((B, D), lambda c: (0, 0)),
        ],
        out_specs=pl.BlockSpec((B, K), lambda c: (0, 0)),
        out_shape=jax.ShapeDtypeStruct((B, K), jnp.int32),
        scratch_shapes=[
            pltpu.VMEM((B, K), jnp.float32),
            pltpu.VMEM((B, K), jnp.int32),
        ],
    )(trainT_p, test_x)


def _gather_mean(train_y, idx_flat):
    mesh = plsc.VectorSubcoreMesh(core_axis_name="c", subcore_axis_name="s")

    @functools.partial(
        pl.kernel,
        mesh=mesh,
        out_type=jax.ShapeDtypeStruct((B, D), jnp.float32),
        scratch_types=[
            pltpu.VMEM((IW,), jnp.int32),
            pltpu.VMEM((IW, D), jnp.float32),
            pltpu.VMEM((QW, D), jnp.float32),
            pltpu.SemaphoreType.DMA,
        ],
        compiler_params=pltpu.CompilerParams(use_tc_tiling_on_sc=False),
    )
    def k(y_hbm, idx_hbm, out_hbm, idx_v, rows_v, acc_v, sem):
        wid = lax.axis_index("s") * 2 + lax.axis_index("c")
        base = wid * IW
        pltpu.sync_copy(idx_hbm.at[pl.ds(base, IW)], idx_v)
        pltpu.async_copy(y_hbm.at[idx_v], rows_v, sem).wait()

        def q_body(q, carry):
            acc = rows_v[q * K]
            for j in range(1, K):
                acc = acc + rows_v[q * K + j]
            acc_v[q] = acc * (1.0 / K)
            return carry

        lax.fori_loop(0, QW, q_body, 0)
        pltpu.sync_copy(acc_v, out_hbm.at[pl.ds(wid * QW, QW)])

    return k(train_y, idx_flat)


def kernel(train_x, train_y, test_x):
    trainT = jnp.pad(train_x, ((0, NP - N), (0, 0))).T
    idx = _topk_call(trainT, test_x)
    return _gather_mean(train_y, idx.reshape(B * K))
